# Initial kernel scaffold; baseline (speedup 1.0000x reference)
import jax
import jax.numpy as jnp
from jax.experimental import pallas as pl


def kernel(x, edge_index, W1, b1, W2, b2, W3, b3, Wfc, bfc):
    raise NotImplementedError("write your pallas kernel here")


# speedup vs baseline: 12.0819x; 12.0819x over previous
import functools

import jax
import jax.numpy as jnp
from jax import lax
from jax.experimental import pallas as pl
from jax.experimental.pallas import tpu as pltpu
from jax.experimental.pallas import tpu_sc as plsc

NC = 2
NS = 16
NW = NC * NS
C = 128


def _edge_agg_kernel(n_acc, n_chunks, width, with_gather):
    rows_per_tile = n_acc // NS
    assert rows_per_tile % C == 0
    mesh = plsc.VectorSubcoreMesh(core_axis_name="c", subcore_axis_name="s")

    scratch = [
        pltpu.VMEM((n_chunks, C), jnp.int32),
        pltpu.VMEM((C, width), jnp.float32),
        pltpu.VMEM_SHARED((n_acc, width), jnp.float32),
        pltpu.SemaphoreType.DMA,
    ]
    if with_gather:
        scratch.insert(0, pltpu.VMEM((n_chunks, C), jnp.int32))

    def body(*refs):
        if with_gather:
            g_hbm, src_hbm, dst_hbm, out_hbm, src_v, dst_v, buf, acc, sem = refs
        else:
            dst_hbm, out_hbm, dst_v, buf, acc, sem = refs
        cid = lax.axis_index("c")
        sid = lax.axis_index("s")
        wid = cid * NS + sid
        row0 = sid * rows_per_tile

        def zrow(j, carry):
            for k in range(width // 16):
                buf[j, pl.ds(k * 16, 16)] = jnp.zeros((16,), jnp.float32)
            return carry
        lax.fori_loop(0, C, zrow, 0)
        for b in range(rows_per_tile // C):
            pltpu.sync_copy(buf, acc.at[pl.ds(row0 + b * C, C)])

        pltpu.sync_copy(dst_hbm.at[wid], dst_v)
        if with_gather:
            pltpu.sync_copy(src_hbm.at[wid], src_v)
        else:
            def orow(j, carry):
                buf[j, pl.ds(0, 16)] = jnp.ones((16,), jnp.float32)
                return carry
            lax.fori_loop(0, C, orow, 0)

        plsc.subcore_barrier()

        def chunk(j, carry):
            if with_gather:
                pltpu.async_copy(g_hbm.at[src_v.at[j]], buf, sem).wait()
            pltpu.sync_copy(buf, acc.at[dst_v.at[j]], add=True)
            return carry
        lax.fori_loop(0, n_chunks, chunk, 0)

        plsc.subcore_barrier()

        for b in range(rows_per_tile // C):
            r = row0 + b * C
            pltpu.sync_copy(acc.at[pl.ds(r, C)], buf)
            pltpu.sync_copy(buf, out_hbm.at[cid, pl.ds(r, C)])

    return pl.kernel(
        body,
        out_type=jax.ShapeDtypeStruct((NC, n_acc, width), jnp.float32),
        mesh=mesh,
        scratch_types=scratch,
    )


def kernel(x, edge_index, W1, b1, W2, b2, W3, b3, Wfc, bfc):
    n, d_in = x.shape
    d_hid = W1.shape[1]
    n_cls = Wfc.shape[1]
    e = edge_index.shape[1]

    n_acc = -(-(n + 1) // (NS * C)) * (NS * C)
    ept = -(-(e // NW) // C) * C
    n_chunks = ept // C
    pad = ept - e // NW

    src = edge_index[0].astype(jnp.int32).reshape(NW, e // NW)
    dst = edge_index[1].astype(jnp.int32).reshape(NW, e // NW)
    src_t = jnp.pad(src, ((0, 0), (0, pad))).reshape(NW, n_chunks, C)
    dst_t = jnp.pad(dst, ((0, 0), (0, pad)), constant_values=n).reshape(
        NW, n_chunks, C)

    deg_pass = _edge_agg_kernel(n_acc, n_chunks, 16, with_gather=False)
    agg_pass = _edge_agg_kernel(n_acc, n_chunks, d_hid, with_gather=True)

    f32 = jnp.float32
    sds = jax.ShapeDtypeStruct

    def tc_prep(degp_ref, x_ref, w_ref, dinv_ref, g_ref):
        deg = degp_ref[0, :n, 0:1] + degp_ref[1, :n, 0:1] + 1.0
        dinv = lax.rsqrt(deg)
        dinv_ref[...] = dinv
        g_ref[...] = dinv * jnp.dot(x_ref[...], w_ref[...],
                                    preferred_element_type=f32)

    def tc_layer(parts_ref, g_ref, dinv_ref, b_ref, w_ref, out_ref):
        agg = parts_ref[0, :n, :] + parts_ref[1, :n, :] + g_ref[...]
        dinv = dinv_ref[...]
        h = jnp.maximum(dinv * agg + b_ref[...], 0.0)
        out_ref[...] = dinv * jnp.dot(h, w_ref[...],
                                      preferred_element_type=f32)

    def tc_final(parts_ref, g_ref, dinv_ref, b_ref, wfc_ref, bfc_ref,
                 h_ref, out_ref):
        agg = parts_ref[0, :n, :] + parts_ref[1, :n, :] + g_ref[...]
        h = jnp.maximum(dinv_ref[...] * agg + b_ref[...], 0.0)
        h_ref[...] = h
        out_ref[...] = jnp.dot(h, wfc_ref[...],
                               preferred_element_type=f32) + bfc_ref[...]

    degp = deg_pass(dst_t)
    dinv, g1 = pl.pallas_call(
        tc_prep, out_shape=(sds((n, 1), f32), sds((n, d_hid), f32)),
    )(degp, x, W1)

    parts1 = agg_pass(g1, src_t, dst_t)
    g2 = pl.pallas_call(
        tc_layer, out_shape=sds((n, d_hid), f32),
    )(parts1, g1, dinv, b1[None, :], W2)

    parts2 = agg_pass(g2, src_t, dst_t)
    g3 = pl.pallas_call(
        tc_layer, out_shape=sds((n, d_hid), f32),
    )(parts2, g2, dinv, b2[None, :], W3)

    parts3 = agg_pass(g3, src_t, dst_t)
    h3, out = pl.pallas_call(
        tc_final, out_shape=(sds((n, d_hid), f32), sds((n, n_cls), f32)),
    )(parts3, g3, dinv, b3[None, :], Wfc, bfc[None, :])
    return (h3, out)
